# baseline (speedup 1.0000x reference)
import functools

import jax
import jax.numpy as jnp
from jax.experimental import pallas as pl
from jax.experimental.pallas import tpu as pltpu

N = 100000
D_IN = 768
D_HID = 512
D_ATT = 256
K = 4
TILE = 2000
NEG = float("-inf")
POS = float("inf")
BIG = 2**30


def _scan_topk(S, gidx, largest: bool):
    cur = S
    sels = []
    for _ in range(K):
        v = jnp.max(cur) if largest else jnp.min(cur)
        sel = jnp.min(jnp.where(cur == v, gidx, BIG))
        cur = jnp.where(gidx == sel, NEG if largest else POS, cur)
        sels.append(sel)
    return sels


def _stream_kernel(h_ref, Wfc_ref, bfc_ref, Wa_ref, ba_ref, Wb_ref, bb_ref,
                   Wc_ref, bc_ref, shift_ref,
                   idx_ref, s_ref, acc_ref,
                   S_ref, svec_ref, accv_ref):
    i = pl.program_id(0)
    nsteps = pl.num_programs(0)

    @pl.when(i == 0)
    def _init():
        svec_ref[...] = jnp.zeros_like(svec_ref)
        accv_ref[...] = jnp.zeros_like(accv_ref)

    h16 = h_ref[...].astype(jnp.bfloat16)
    h1 = jnp.maximum(
        jnp.dot(h16, Wfc_ref[...], preferred_element_type=jnp.float32)
        + bfc_ref[...], 0.0)
    h1_16 = h1.astype(jnp.bfloat16)
    a = jnp.tanh(
        jnp.dot(h1_16, Wa_ref[...], preferred_element_type=jnp.float32)
        + ba_ref[...])
    b = jax.nn.sigmoid(
        jnp.dot(h1_16, Wb_ref[...], preferred_element_type=jnp.float32)
        + bb_ref[...])
    ab16 = (a * b).astype(jnp.bfloat16)
    A2 = jax.lax.dot_general(
        Wc_ref[...], ab16, (((1,), (1,)), ((), ())),
        preferred_element_type=jnp.float32) + bc_ref[0, 0]

    S_ref[pl.ds(i, 1), :] = A2

    w = jnp.exp(A2 - shift_ref[0, 0])
    svec_ref[...] += w
    accv_ref[...] += jnp.dot(w, h1, preferred_element_type=jnp.float32)

    @pl.when(i == nsteps - 1)
    def _epilogue():
        S = S_ref[...]
        gidx = (jax.lax.broadcasted_iota(jnp.int32, S.shape, 0) * TILE
                + jax.lax.broadcasted_iota(jnp.int32, S.shape, 1))
        sels = (_scan_topk(S, gidx, largest=True)
                + _scan_topk(S, gidx, largest=False))
        idx_ref[...] = jnp.stack(sels).reshape(1, 2 * K)
        s_ref[...] = jnp.sum(svec_ref[...]).reshape(1, 1)
        acc_ref[...] = accv_ref[...]


def _gather_kernel(idx_sref, h_row_ref, Wfc_ref, bfc_ref,
                   Wcls_ref, bcls_ref, Wi_ref, bi_ref,
                   s_ref, acc_ref, out_ref, rows_ref):
    j = pl.program_id(0)
    h16 = h_row_ref[0].astype(jnp.bfloat16)
    h1 = jnp.maximum(
        jnp.dot(h16, Wfc_ref[...], preferred_element_type=jnp.float32)
        + bfc_ref[...], 0.0)
    rows_ref[pl.ds(j, 1)] = h1.astype(jnp.bfloat16).reshape(1, 1, D_HID)

    @pl.when(j == 2 * K - 1)
    def _fin():
        M = (acc_ref[...] / s_ref[0, 0]).astype(jnp.bfloat16)
        bag = jnp.dot(M, Wcls_ref[...],
                      preferred_element_type=jnp.float32) + bcls_ref[...]
        inst = jnp.dot(rows_ref[...].reshape(2 * K, D_HID), Wi_ref[...],
                       preferred_element_type=jnp.float32) + bi_ref[...]
        out_ref[...] = jnp.zeros((16, 128), dtype=jnp.float32)
        out_ref[0:1, 0:2] = bag
        out_ref[1:1 + 2 * K, 0:2] = inst


@functools.partial(jax.jit, static_argnames=())
def kernel(h, W_fc, b_fc, Wa, ba, Wb, bb, Wc, bc, Wcls, bcls, Wi, bi):
    nsteps = N // TILE
    const = lambda *_: (0, 0)
    Wfc16 = W_fc.astype(jnp.bfloat16)
    Wc16 = Wc.astype(jnp.bfloat16).reshape(1, D_ATT)
    shift = (jnp.sum(jnp.abs(Wc16.astype(jnp.float32)))
             + jnp.abs(bc[0])).reshape(1, 1)

    idx, s, acc = pl.pallas_call(
        _stream_kernel,
        grid=(nsteps,),
        in_specs=[
            pl.BlockSpec((TILE, D_IN), lambda i: (i, 0)),
            pl.BlockSpec((D_IN, D_HID), const),
            pl.BlockSpec((1, D_HID), const),
            pl.BlockSpec((D_HID, D_ATT), const),
            pl.BlockSpec((1, D_ATT), const),
            pl.BlockSpec((D_HID, D_ATT), const),
            pl.BlockSpec((1, D_ATT), const),
            pl.BlockSpec((1, D_ATT), const),
            pl.BlockSpec((1, 1), const),
            pl.BlockSpec((1, 1), const),
        ],
        out_specs=[
            pl.BlockSpec((1, 2 * K), const),
            pl.BlockSpec((1, 1), const),
            pl.BlockSpec((1, D_HID), const),
        ],
        out_shape=[
            jax.ShapeDtypeStruct((1, 2 * K), jnp.int32),
            jax.ShapeDtypeStruct((1, 1), jnp.float32),
            jax.ShapeDtypeStruct((1, D_HID), jnp.float32),
        ],
        scratch_shapes=[
            pltpu.VMEM((nsteps, TILE), jnp.float32),
            pltpu.VMEM((1, TILE), jnp.float32),
            pltpu.VMEM((1, D_HID), jnp.float32),
        ],
    )(
        h, Wfc16, b_fc.reshape(1, D_HID),
        Wa.astype(jnp.bfloat16), ba.reshape(1, D_ATT),
        Wb.astype(jnp.bfloat16), bb.reshape(1, D_ATT),
        Wc16, bc.reshape(1, 1), shift,
    )

    out = pl.pallas_call(
        _gather_kernel,
        grid_spec=pltpu.PrefetchScalarGridSpec(
            num_scalar_prefetch=1,
            grid=(2 * K,),
            in_specs=[
                pl.BlockSpec((1, 1, D_IN), lambda j, idx_s: (idx_s[j], 0, 0)),
                pl.BlockSpec((D_IN, D_HID), lambda j, idx_s: (0, 0)),
                pl.BlockSpec((1, D_HID), lambda j, idx_s: (0, 0)),
                pl.BlockSpec((D_HID, 2), lambda j, idx_s: (0, 0)),
                pl.BlockSpec((1, 2), lambda j, idx_s: (0, 0)),
                pl.BlockSpec((D_HID, 2), lambda j, idx_s: (0, 0)),
                pl.BlockSpec((1, 2), lambda j, idx_s: (0, 0)),
                pl.BlockSpec((1, 1), lambda j, idx_s: (0, 0)),
                pl.BlockSpec((1, D_HID), lambda j, idx_s: (0, 0)),
            ],
            out_specs=pl.BlockSpec((16, 128), lambda j, idx_s: (0, 0)),
            scratch_shapes=[
                pltpu.VMEM((2 * K, 1, D_HID), jnp.bfloat16),
            ],
        ),
        out_shape=jax.ShapeDtypeStruct((16, 128), jnp.float32),
    )(
        idx.reshape(2 * K), h.reshape(N, 1, D_IN), Wfc16,
        b_fc.reshape(1, D_HID),
        Wcls.astype(jnp.bfloat16), bcls.reshape(1, 2),
        Wi.astype(jnp.bfloat16), bi.reshape(1, 2),
        s, acc,
    )
    return out[0:1 + 2 * K, 0:2]


# speedup vs baseline: 4.6634x; 4.6634x over previous
import functools

import jax
import jax.numpy as jnp
from jax.experimental import pallas as pl
from jax.experimental.pallas import tpu as pltpu

N = 100000
D_IN = 768
D_HID = 512
D_ATT = 256
K = 4
TILE = 2000
NEG = float("-inf")
POS = float("inf")
BIG = 2**30


def _scan_topk(S, gidx, largest: bool):
    cur = S
    sels = []
    for _ in range(K):
        v = jnp.max(cur) if largest else jnp.min(cur)
        sel = jnp.min(jnp.where(cur == v, gidx, BIG))
        cur = jnp.where(gidx == sel, NEG if largest else POS, cur)
        sels.append(sel)
    return sels


def _stream_kernel(h_ref, Wfc_ref, bfc_ref, Wa_ref, ba_ref, Wb_ref, bb_ref,
                   Wc_ref, bc_ref, shift_ref,
                   idx_ref, s_ref, acc_ref,
                   S_ref, svec_ref, accv_ref):
    i = pl.program_id(0)
    nsteps = pl.num_programs(0)

    @pl.when(i == 0)
    def _init():
        svec_ref[...] = jnp.zeros_like(svec_ref)
        accv_ref[...] = jnp.zeros_like(accv_ref)

    h16 = h_ref[...].astype(jnp.bfloat16)
    h1 = jnp.maximum(
        jnp.dot(h16, Wfc_ref[...], preferred_element_type=jnp.float32)
        + bfc_ref[...], 0.0)
    h1_16 = h1.astype(jnp.bfloat16)
    a = jnp.tanh(
        jnp.dot(h1_16, Wa_ref[...], preferred_element_type=jnp.float32)
        + ba_ref[...])
    b = jax.nn.sigmoid(
        jnp.dot(h1_16, Wb_ref[...], preferred_element_type=jnp.float32)
        + bb_ref[...])
    ab16 = (a * b).astype(jnp.bfloat16)
    A2 = jax.lax.dot_general(
        Wc_ref[...], ab16, (((1,), (1,)), ((), ())),
        preferred_element_type=jnp.float32) + bc_ref[0, 0]

    S_ref[pl.ds(i, 1), :] = A2

    w = jnp.exp(A2 - shift_ref[0, 0])
    svec_ref[...] += w
    accv_ref[...] += jnp.dot(w, h1, preferred_element_type=jnp.float32)

    @pl.when(i == nsteps - 1)
    def _epilogue():
        S = S_ref[...]
        gidx = (jax.lax.broadcasted_iota(jnp.int32, S.shape, 0) * TILE
                + jax.lax.broadcasted_iota(jnp.int32, S.shape, 1))
        sels = (_scan_topk(S, gidx, largest=True)
                + _scan_topk(S, gidx, largest=False))
        idx_ref[...] = jnp.stack(sels).reshape(1, 2 * K)
        s_ref[...] = jnp.sum(svec_ref[...]).reshape(1, 1)
        acc_ref[...] = accv_ref[...]


def _gather_kernel(idx_sref, h_row_ref, Wfc_ref, bfc_ref,
                   Wcls_ref, bcls_ref, Wi_ref, bi_ref,
                   s_ref, acc_ref, out_ref, rows_ref):
    j = pl.program_id(0)
    r = idx_sref[j] % 8
    rowsel = jax.lax.broadcasted_iota(jnp.int32, (8, 1), 0) == r
    h_row = jnp.sum(jnp.where(rowsel, h_row_ref[...], 0.0),
                    axis=0, keepdims=True)
    h16 = h_row.astype(jnp.bfloat16)
    h1 = jnp.maximum(
        jnp.dot(h16, Wfc_ref[...], preferred_element_type=jnp.float32)
        + bfc_ref[...], 0.0)
    rows_ref[pl.ds(j, 1)] = h1.astype(jnp.bfloat16).reshape(1, 1, D_HID)

    @pl.when(j == 2 * K - 1)
    def _fin():
        M = (acc_ref[...] / s_ref[0, 0]).astype(jnp.bfloat16)
        bag = jnp.dot(M, Wcls_ref[...],
                      preferred_element_type=jnp.float32) + bcls_ref[...]
        inst = jnp.dot(rows_ref[...].reshape(2 * K, D_HID), Wi_ref[...],
                       preferred_element_type=jnp.float32) + bi_ref[...]
        out_ref[...] = jnp.zeros((16, 128), dtype=jnp.float32)
        out_ref[0:1, 0:2] = bag
        out_ref[1:1 + 2 * K, 0:2] = inst


@functools.partial(jax.jit, static_argnames=())
def kernel(h, W_fc, b_fc, Wa, ba, Wb, bb, Wc, bc, Wcls, bcls, Wi, bi):
    nsteps = N // TILE
    const = lambda *_: (0, 0)
    Wfc16 = W_fc.astype(jnp.bfloat16)
    Wc16 = Wc.astype(jnp.bfloat16).reshape(1, D_ATT)
    shift = (jnp.sum(jnp.abs(Wc16.astype(jnp.float32)))
             + jnp.abs(bc[0])).reshape(1, 1)

    idx, s, acc = pl.pallas_call(
        _stream_kernel,
        grid=(nsteps,),
        in_specs=[
            pl.BlockSpec((TILE, D_IN), lambda i: (i, 0)),
            pl.BlockSpec((D_IN, D_HID), const),
            pl.BlockSpec((1, D_HID), const),
            pl.BlockSpec((D_HID, D_ATT), const),
            pl.BlockSpec((1, D_ATT), const),
            pl.BlockSpec((D_HID, D_ATT), const),
            pl.BlockSpec((1, D_ATT), const),
            pl.BlockSpec((1, D_ATT), const),
            pl.BlockSpec((1, 1), const),
            pl.BlockSpec((1, 1), const),
        ],
        out_specs=[
            pl.BlockSpec((1, 2 * K), const),
            pl.BlockSpec((1, 1), const),
            pl.BlockSpec((1, D_HID), const),
        ],
        out_shape=[
            jax.ShapeDtypeStruct((1, 2 * K), jnp.int32),
            jax.ShapeDtypeStruct((1, 1), jnp.float32),
            jax.ShapeDtypeStruct((1, D_HID), jnp.float32),
        ],
        scratch_shapes=[
            pltpu.VMEM((nsteps, TILE), jnp.float32),
            pltpu.VMEM((1, TILE), jnp.float32),
            pltpu.VMEM((1, D_HID), jnp.float32),
        ],
    )(
        h, Wfc16, b_fc.reshape(1, D_HID),
        Wa.astype(jnp.bfloat16), ba.reshape(1, D_ATT),
        Wb.astype(jnp.bfloat16), bb.reshape(1, D_ATT),
        Wc16, bc.reshape(1, 1), shift,
    )

    out = pl.pallas_call(
        _gather_kernel,
        grid_spec=pltpu.PrefetchScalarGridSpec(
            num_scalar_prefetch=1,
            grid=(2 * K,),
            in_specs=[
                pl.BlockSpec((8, D_IN), lambda j, idx_s: (idx_s[j] // 8, 0)),
                pl.BlockSpec((D_IN, D_HID), lambda j, idx_s: (0, 0)),
                pl.BlockSpec((1, D_HID), lambda j, idx_s: (0, 0)),
                pl.BlockSpec((D_HID, 2), lambda j, idx_s: (0, 0)),
                pl.BlockSpec((1, 2), lambda j, idx_s: (0, 0)),
                pl.BlockSpec((D_HID, 2), lambda j, idx_s: (0, 0)),
                pl.BlockSpec((1, 2), lambda j, idx_s: (0, 0)),
                pl.BlockSpec((1, 1), lambda j, idx_s: (0, 0)),
                pl.BlockSpec((1, D_HID), lambda j, idx_s: (0, 0)),
            ],
            out_specs=pl.BlockSpec((16, 128), lambda j, idx_s: (0, 0)),
            scratch_shapes=[
                pltpu.VMEM((2 * K, 1, D_HID), jnp.bfloat16),
            ],
        ),
        out_shape=jax.ShapeDtypeStruct((16, 128), jnp.float32),
    )(
        idx.reshape(2 * K), h, Wfc16,
        b_fc.reshape(1, D_HID),
        Wcls.astype(jnp.bfloat16), bcls.reshape(1, 2),
        Wi.astype(jnp.bfloat16), bi.reshape(1, 2),
        s, acc,
    )
    return out[0:1 + 2 * K, 0:2]


# speedup vs baseline: 4.7036x; 1.0086x over previous
import functools

import jax
import jax.numpy as jnp
from jax.experimental import pallas as pl
from jax.experimental.pallas import tpu as pltpu

N = 100000
D_IN = 768
D_HID = 512
D_ATT = 256
K = 4
TILE = 2000
NEG = float("-inf")
POS = float("inf")
BIG = 2**30


def _scan_topk(S, gidx, largest: bool):
    cur = S
    sels = []
    for _ in range(K):
        v = jnp.max(cur) if largest else jnp.min(cur)
        sel = jnp.min(jnp.where(cur == v, gidx, BIG))
        cur = jnp.where(gidx == sel, NEG if largest else POS, cur)
        sels.append(sel)
    return sels


def _stream_kernel(h_ref, Wfc_ref, bfc_ref, Wa_ref, ba_ref, Wb_ref, bb_ref,
                   Wc_ref, bc_ref, shift_ref,
                   idx_ref, s_ref, acc_ref,
                   S_ref, svec_ref, accv_ref):
    i = pl.program_id(0)
    nsteps = pl.num_programs(0)

    @pl.when(i == 0)
    def _init():
        svec_ref[...] = jnp.zeros_like(svec_ref)
        accv_ref[...] = jnp.zeros_like(accv_ref)

    h1_16 = jnp.maximum(
        jnp.dot(h_ref[...], Wfc_ref[...],
                precision=jax.lax.Precision.DEFAULT,
                preferred_element_type=jnp.float32)
        + bfc_ref[...], 0.0).astype(jnp.bfloat16)
    a = jnp.tanh(
        jnp.dot(h1_16, Wa_ref[...], preferred_element_type=jnp.float32)
        + ba_ref[...])
    b = jax.nn.sigmoid(
        jnp.dot(h1_16, Wb_ref[...], preferred_element_type=jnp.float32)
        + bb_ref[...])
    A2 = jax.lax.dot_general(
        Wc_ref[...], a * b, (((1,), (1,)), ((), ())),
        precision=jax.lax.Precision.DEFAULT,
        preferred_element_type=jnp.float32) + bc_ref[0, 0]

    S_ref[pl.ds(i, 1), :] = A2

    w = jnp.exp(A2 - shift_ref[0, 0])
    svec_ref[...] += w
    accv_ref[...] += jnp.dot(w.astype(jnp.bfloat16), h1_16,
                             preferred_element_type=jnp.float32)

    @pl.when(i == nsteps - 1)
    def _epilogue():
        S = S_ref[...]
        gidx = (jax.lax.broadcasted_iota(jnp.int32, S.shape, 0) * TILE
                + jax.lax.broadcasted_iota(jnp.int32, S.shape, 1))
        sels = (_scan_topk(S, gidx, largest=True)
                + _scan_topk(S, gidx, largest=False))
        idx_ref[...] = jnp.stack(sels).reshape(1, 2 * K)
        s_ref[...] = jnp.sum(svec_ref[...]).reshape(1, 1)
        acc_ref[...] = accv_ref[...]


def _gather_kernel(idx_sref, h_row_ref, Wfc_ref, bfc_ref,
                   Wcls_ref, bcls_ref, Wi_ref, bi_ref,
                   s_ref, acc_ref, out_ref, rows_ref):
    j = pl.program_id(0)
    r = idx_sref[j] % 8
    rowsel = jax.lax.broadcasted_iota(jnp.int32, (8, 1), 0) == r
    h_row = jnp.sum(jnp.where(rowsel, h_row_ref[...], 0.0),
                    axis=0, keepdims=True)
    h16 = h_row.astype(jnp.bfloat16)
    h1 = jnp.maximum(
        jnp.dot(h16, Wfc_ref[...], preferred_element_type=jnp.float32)
        + bfc_ref[...], 0.0)
    rows_ref[pl.ds(j, 1)] = h1.astype(jnp.bfloat16).reshape(1, 1, D_HID)

    @pl.when(j == 2 * K - 1)
    def _fin():
        M = (acc_ref[...] / s_ref[0, 0]).astype(jnp.bfloat16)
        bag = jnp.dot(M, Wcls_ref[...],
                      preferred_element_type=jnp.float32) + bcls_ref[...]
        inst = jnp.dot(rows_ref[...].reshape(2 * K, D_HID), Wi_ref[...],
                       preferred_element_type=jnp.float32) + bi_ref[...]
        out_ref[...] = jnp.zeros((16, 128), dtype=jnp.float32)
        out_ref[0:1, 0:2] = bag
        out_ref[1:1 + 2 * K, 0:2] = inst


@functools.partial(jax.jit, static_argnames=())
def kernel(h, W_fc, b_fc, Wa, ba, Wb, bb, Wc, bc, Wcls, bcls, Wi, bi):
    nsteps = N // TILE
    const = lambda *_: (0, 0)
    Wfc16 = W_fc.astype(jnp.bfloat16)
    Wc16 = Wc.astype(jnp.bfloat16).reshape(1, D_ATT)
    shift = (jnp.sum(jnp.abs(Wc16.astype(jnp.float32)))
             + jnp.abs(bc[0])).reshape(1, 1)

    idx, s, acc = pl.pallas_call(
        _stream_kernel,
        grid=(nsteps,),
        in_specs=[
            pl.BlockSpec((TILE, D_IN), lambda i: (i, 0)),
            pl.BlockSpec((D_IN, D_HID), const),
            pl.BlockSpec((1, D_HID), const),
            pl.BlockSpec((D_HID, D_ATT), const),
            pl.BlockSpec((1, D_ATT), const),
            pl.BlockSpec((D_HID, D_ATT), const),
            pl.BlockSpec((1, D_ATT), const),
            pl.BlockSpec((1, D_ATT), const),
            pl.BlockSpec((1, 1), const),
            pl.BlockSpec((1, 1), const),
        ],
        out_specs=[
            pl.BlockSpec((1, 2 * K), const),
            pl.BlockSpec((1, 1), const),
            pl.BlockSpec((1, D_HID), const),
        ],
        out_shape=[
            jax.ShapeDtypeStruct((1, 2 * K), jnp.int32),
            jax.ShapeDtypeStruct((1, 1), jnp.float32),
            jax.ShapeDtypeStruct((1, D_HID), jnp.float32),
        ],
        scratch_shapes=[
            pltpu.VMEM((nsteps, TILE), jnp.float32),
            pltpu.VMEM((1, TILE), jnp.float32),
            pltpu.VMEM((1, D_HID), jnp.float32),
        ],
    )(
        h, W_fc, b_fc.reshape(1, D_HID),
        Wa.astype(jnp.bfloat16), ba.reshape(1, D_ATT),
        Wb.astype(jnp.bfloat16), bb.reshape(1, D_ATT),
        Wc.reshape(1, D_ATT), bc.reshape(1, 1), shift,
    )

    out = pl.pallas_call(
        _gather_kernel,
        grid_spec=pltpu.PrefetchScalarGridSpec(
            num_scalar_prefetch=1,
            grid=(2 * K,),
            in_specs=[
                pl.BlockSpec((8, D_IN), lambda j, idx_s: (idx_s[j] // 8, 0)),
                pl.BlockSpec((D_IN, D_HID), lambda j, idx_s: (0, 0)),
                pl.BlockSpec((1, D_HID), lambda j, idx_s: (0, 0)),
                pl.BlockSpec((D_HID, 2), lambda j, idx_s: (0, 0)),
                pl.BlockSpec((1, 2), lambda j, idx_s: (0, 0)),
                pl.BlockSpec((D_HID, 2), lambda j, idx_s: (0, 0)),
                pl.BlockSpec((1, 2), lambda j, idx_s: (0, 0)),
                pl.BlockSpec((1, 1), lambda j, idx_s: (0, 0)),
                pl.BlockSpec((1, D_HID), lambda j, idx_s: (0, 0)),
            ],
            out_specs=pl.BlockSpec((16, 128), lambda j, idx_s: (0, 0)),
            scratch_shapes=[
                pltpu.VMEM((2 * K, 1, D_HID), jnp.bfloat16),
            ],
        ),
        out_shape=jax.ShapeDtypeStruct((16, 128), jnp.float32),
    )(
        idx.reshape(2 * K), h, Wfc16,
        b_fc.reshape(1, D_HID),
        Wcls.astype(jnp.bfloat16), bcls.reshape(1, 2),
        Wi.astype(jnp.bfloat16), bi.reshape(1, 2),
        s, acc,
    )
    return out[0:1 + 2 * K, 0:2]


# speedup vs baseline: 4.7389x; 1.0075x over previous
import functools

import jax
import jax.numpy as jnp
from jax.experimental import pallas as pl
from jax.experimental.pallas import tpu as pltpu

N = 100000
D_IN = 768
D_HID = 512
D_ATT = 256
K = 4
TILE = 2000
NEG = float("-inf")
POS = float("inf")
BIG = 2**30
DEFAULT = jax.lax.Precision.DEFAULT


def _scan_topk(S, gidx, largest: bool):
    cur = S
    sels = []
    for _ in range(K):
        v = jnp.max(cur) if largest else jnp.min(cur)
        sel = jnp.min(jnp.where(cur == v, gidx, BIG))
        cur = jnp.where(gidx == sel, NEG if largest else POS, cur)
        sels.append(sel)
    return sels


def _stream_kernel(h_ref, Wfc_ref, bfc_ref, Wa_ref, ba_ref, Wb_ref, bb_ref,
                   Wc_ref, bc_ref, shift_ref,
                   idx_ref, s_ref, acc_ref,
                   S_ref, svec_ref, accv_ref):
    i = pl.program_id(0)
    nsteps = pl.num_programs(0)

    @pl.when(i == 0)
    def _init():
        svec_ref[...] = jnp.zeros_like(svec_ref)
        accv_ref[...] = jnp.zeros_like(accv_ref)

    h1 = jnp.maximum(
        jnp.dot(h_ref[...], Wfc_ref[...], precision=DEFAULT,
                preferred_element_type=jnp.float32)
        + bfc_ref[...], 0.0)
    a = jnp.tanh(
        jnp.dot(h1, Wa_ref[...], precision=DEFAULT,
                preferred_element_type=jnp.float32)
        + ba_ref[...])
    b = jax.nn.sigmoid(
        jnp.dot(h1, Wb_ref[...], precision=DEFAULT,
                preferred_element_type=jnp.float32)
        + bb_ref[...])
    A2 = jax.lax.dot_general(
        Wc_ref[...], a * b, (((0,), (1,)), ((), ())),
        precision=DEFAULT,
        preferred_element_type=jnp.float32) + bc_ref[0, 0]

    S_ref[pl.ds(i, 1), :] = A2

    w = jnp.exp(A2 - shift_ref[0, 0])
    svec_ref[...] += w
    accv_ref[...] += jnp.dot(w, h1, precision=DEFAULT,
                             preferred_element_type=jnp.float32)

    @pl.when(i == nsteps - 1)
    def _epilogue():
        S = S_ref[...]
        gidx = (jax.lax.broadcasted_iota(jnp.int32, S.shape, 0) * TILE
                + jax.lax.broadcasted_iota(jnp.int32, S.shape, 1))
        sels = (_scan_topk(S, gidx, largest=True)
                + _scan_topk(S, gidx, largest=False))
        idx_ref[...] = jnp.stack(sels).reshape(1, 2 * K)
        s_ref[...] = jnp.sum(svec_ref[...]).reshape(1, 1)
        acc_ref[...] = accv_ref[...]


def _gather_kernel(idx_sref, h_row_ref, Wfc_ref, bfc_ref,
                   Wcls_ref, bcls_ref, Wi_ref, bi_ref,
                   s_ref, acc_ref, out_ref, rows_ref):
    j = pl.program_id(0)
    r = idx_sref[j] % 8
    rowsel = jax.lax.broadcasted_iota(jnp.int32, (8, 1), 0) == r
    h_row = jnp.sum(jnp.where(rowsel, h_row_ref[...], 0.0),
                    axis=0, keepdims=True)
    h1 = jnp.maximum(
        jnp.dot(h_row, Wfc_ref[...], precision=DEFAULT,
                preferred_element_type=jnp.float32)
        + bfc_ref[...], 0.0)
    rows_ref[pl.ds(j, 1)] = h1.reshape(1, 1, D_HID)

    @pl.when(j == 2 * K - 1)
    def _fin():
        M = acc_ref[...] / s_ref[0, 0]
        bag = jnp.dot(M, Wcls_ref[...], precision=DEFAULT,
                      preferred_element_type=jnp.float32) + bcls_ref[...]
        inst = jnp.dot(rows_ref[...].reshape(2 * K, D_HID), Wi_ref[...],
                       precision=DEFAULT,
                       preferred_element_type=jnp.float32) + bi_ref[...]
        out_ref[...] = jnp.zeros((16, 128), dtype=jnp.float32)
        out_ref[0:1, 0:2] = bag
        out_ref[1:1 + 2 * K, 0:2] = inst


@functools.partial(jax.jit, static_argnames=())
def kernel(h, W_fc, b_fc, Wa, ba, Wb, bb, Wc, bc, Wcls, bcls, Wi, bi):
    nsteps = N // TILE
    const = lambda *_: (0, 0)
    shift = (jnp.sum(jnp.abs(Wc.astype(jnp.bfloat16).astype(jnp.float32)))
             + jnp.abs(bc[0])).reshape(1, 1)

    idx, s, acc = pl.pallas_call(
        _stream_kernel,
        grid=(nsteps,),
        in_specs=[
            pl.BlockSpec((TILE, D_IN), lambda i: (i, 0)),
            pl.BlockSpec((D_IN, D_HID), const),
            pl.BlockSpec((1, D_HID), const),
            pl.BlockSpec((D_HID, D_ATT), const),
            pl.BlockSpec((1, D_ATT), const),
            pl.BlockSpec((D_HID, D_ATT), const),
            pl.BlockSpec((1, D_ATT), const),
            pl.BlockSpec((D_ATT, 1), const),
            pl.BlockSpec((1, 1), const),
            pl.BlockSpec((1, 1), const),
        ],
        out_specs=[
            pl.BlockSpec((1, 2 * K), const),
            pl.BlockSpec((1, 1), const),
            pl.BlockSpec((1, D_HID), const),
        ],
        out_shape=[
            jax.ShapeDtypeStruct((1, 2 * K), jnp.int32),
            jax.ShapeDtypeStruct((1, 1), jnp.float32),
            jax.ShapeDtypeStruct((1, D_HID), jnp.float32),
        ],
        scratch_shapes=[
            pltpu.VMEM((nsteps, TILE), jnp.float32),
            pltpu.VMEM((1, TILE), jnp.float32),
            pltpu.VMEM((1, D_HID), jnp.float32),
        ],
    )(
        h, W_fc, b_fc.reshape(1, D_HID),
        Wa, ba.reshape(1, D_ATT),
        Wb, bb.reshape(1, D_ATT),
        Wc, bc.reshape(1, 1), shift,
    )

    out = pl.pallas_call(
        _gather_kernel,
        grid_spec=pltpu.PrefetchScalarGridSpec(
            num_scalar_prefetch=1,
            grid=(2 * K,),
            in_specs=[
                pl.BlockSpec((8, D_IN), lambda j, idx_s: (idx_s[j] // 8, 0)),
                pl.BlockSpec((D_IN, D_HID), lambda j, idx_s: (0, 0)),
                pl.BlockSpec((1, D_HID), lambda j, idx_s: (0, 0)),
                pl.BlockSpec((D_HID, 2), lambda j, idx_s: (0, 0)),
                pl.BlockSpec((1, 2), lambda j, idx_s: (0, 0)),
                pl.BlockSpec((D_HID, 2), lambda j, idx_s: (0, 0)),
                pl.BlockSpec((1, 2), lambda j, idx_s: (0, 0)),
                pl.BlockSpec((1, 1), lambda j, idx_s: (0, 0)),
                pl.BlockSpec((1, D_HID), lambda j, idx_s: (0, 0)),
            ],
            out_specs=pl.BlockSpec((16, 128), lambda j, idx_s: (0, 0)),
            scratch_shapes=[
                pltpu.VMEM((2 * K, 1, D_HID), jnp.float32),
            ],
        ),
        out_shape=jax.ShapeDtypeStruct((16, 128), jnp.float32),
    )(
        idx.reshape(2 * K), h, W_fc,
        b_fc.reshape(1, D_HID),
        Wcls, bcls.reshape(1, 2),
        Wi, bi.reshape(1, 2),
        s, acc,
    )
    return out[0:1 + 2 * K, 0:2]
